# baseline (speedup 1.0000x reference)
import functools

import jax
import jax.numpy as jnp
from jax.experimental import pallas as pl
from jax.experimental.pallas import tpu as pltpu

STRONG = 255.0
WEAK = 25.0
LOW_T = 0.05
HIGH_T = 0.15

H = W = 512
NEG = -2.0


def _sr(a, s, fill=0.0):
    z = jnp.full(a.shape[:-1] + (s,), fill, a.dtype)
    return jnp.concatenate([z, a[..., : a.shape[-1] - s]], axis=-1)


def _sl(a, s, fill=0.0):
    z = jnp.full(a.shape[:-1] + (s,), fill, a.dtype)
    return jnp.concatenate([a[..., s:], z], axis=-1)


def _sd(a, s, fill=0.0):
    z = jnp.full((s,) + a.shape[1:], fill, a.dtype)
    return jnp.concatenate([z, a[: a.shape[0] - s]], axis=0)


def _su(a, s, fill=0.0):
    z = jnp.full((s,) + a.shape[1:], fill, a.dtype)
    return jnp.concatenate([a[s:], z], axis=0)


def _hyst_kernel(x_ref, o_ref, w_ref, b_ref, p_ref):
    x = x_ref[...]
    hi = jnp.max(x) * HIGH_T
    lo = hi * LOW_T
    strong = (x > hi).astype(jnp.float32)
    weakb = jnp.logical_and(x >= lo, x <= hi)

    col = jax.lax.broadcasted_iota(jnp.int32, (H, W), 1)
    row = jax.lax.broadcasted_iota(jnp.int32, (H, W), 0)
    incol = jnp.logical_and(col > 0, col < W - 1)
    w = jnp.logical_and(weakb, incol).astype(jnp.float32)
    w_ref[...] = w

    h3 = jnp.maximum(strong, jnp.maximum(_sr(strong, 1), _sl(strong, 1)))
    d = jnp.maximum(
        jnp.maximum(_sr(strong, 1), _sl(strong, 1)),
        jnp.maximum(_sd(h3, 1), _su(h3, 1)),
    )

    colf = col.astype(jnp.float32)
    b = jnp.where(w > 0.0, -1.0, colf)
    sd_idx = jnp.where(jnp.logical_and(w > 0.0, d > 0.0), colf, NEG)
    for s in (1, 2, 4, 8, 16, 32, 64, 128, 256):
        b = jnp.maximum(b, _sr(b, s, NEG))
        sd_idx = jnp.maximum(sd_idx, _sr(sd_idx, s, NEG))
    beff = jnp.where(sd_idx > b, -600.0, b)
    b_ref[...] = beff
    p_ref[...] = jnp.zeros((H, W), jnp.float32)

    iota1 = colf[0:1, :] + 1.0

    def row_body(i, p_prev):
        c = jnp.maximum(p_prev, jnp.maximum(_sr(p_prev, 1), _sl(p_prev, 1)))
        cp = w_ref[pl.ds(i, 1), :] * c
        m = cp * iota1 - 1.0
        for s in (1, 2, 4, 8, 16, 32, 64, 128, 256):
            m = jnp.maximum(m, _sr(m, s, NEG))
        p_new = (m > b_ref[pl.ds(i, 1), :]).astype(jnp.float32)
        p_ref[pl.ds(i, 1), :] = p_new
        return p_new

    jax.lax.fori_loop(1, H - 1, row_body, jnp.zeros((1, W), jnp.float32))

    p = p_ref[...]
    tx = jnp.where(weakb, WEAK, jnp.where(x >= hi, STRONG, 0.0))
    interior = jnp.logical_and(jnp.logical_and(row > 0, row < H - 1), incol)
    o_ref[...] = jnp.where(
        jnp.logical_and(weakb, interior),
        jnp.where(p > 0.5, STRONG, 0.0),
        tx,
    )


@functools.partial(jax.jit)
def kernel(img):
    x = img.reshape(H, W)
    out = pl.pallas_call(
        _hyst_kernel,
        out_shape=jax.ShapeDtypeStruct((H, W), jnp.float32),
        scratch_shapes=[
            pltpu.VMEM((H, W), jnp.float32),
            pltpu.VMEM((H, W), jnp.float32),
            pltpu.VMEM((H, W), jnp.float32),
        ],
    )(x)
    return out[None, None, :, :]


# speedup vs baseline: 13.6287x; 13.6287x over previous
import functools

import jax
import jax.numpy as jnp
from jax.experimental import pallas as pl
from jax.experimental.pallas import tpu as pltpu

STRONG = 255.0
WEAK = 25.0
LOW_T = 0.05
HIGH_T = 0.15

H = W = 512
NW = W // 32
FULL = -1


def _sr(a, s):
    z = jnp.zeros(a.shape[:-1] + (s,), a.dtype)
    return jnp.concatenate([z, a[..., : a.shape[-1] - s]], axis=-1)


def _sl(a, s):
    z = jnp.zeros(a.shape[:-1] + (s,), a.dtype)
    return jnp.concatenate([a[..., s:], z], axis=-1)


def _sd(a, s):
    z = jnp.zeros((s,) + a.shape[1:], a.dtype)
    return jnp.concatenate([z, a[: a.shape[0] - s]], axis=0)


def _su(a, s):
    z = jnp.zeros((s,) + a.shape[1:], a.dtype)
    return jnp.concatenate([a[s:], z], axis=0)


def _fill(t, s):
    return t & (~(t + s) | s)


def _hyst_kernel(x_ref, o_ref, wtab_ref, dtab_ref, ptab_ref):
    x = x_ref[...]
    hi = jnp.max(x) * HIGH_T
    lo = hi * LOW_T
    strongb = x > hi
    weakb = jnp.logical_and(x >= lo, x <= hi)

    col = jax.lax.broadcasted_iota(jnp.int32, (H, W), 1)
    row = jax.lax.broadcasted_iota(jnp.int32, (H, W), 0)
    incol = jnp.logical_and(col > 0, col < W - 1)
    inrow = jnp.logical_and(row > 0, row < H - 1)
    wf = (jnp.logical_and(jnp.logical_and(weakb, incol), inrow)
          ).astype(jnp.float32)
    sf = strongb.astype(jnp.float32)

    jj = jax.lax.broadcasted_iota(jnp.int32, (W, NW), 0)
    ll = jax.lax.broadcasted_iota(jnp.int32, (W, NW), 1)
    inw = (jj // 32) == ll
    bit = jj % 32
    pk_lo = jnp.where(jnp.logical_and(inw, bit < 16),
                      jax.lax.shift_left(1, bit), 0).astype(jnp.bfloat16)
    pk_hi = jnp.where(jnp.logical_and(inw, bit >= 16),
                      jax.lax.shift_left(1, bit - 16), 0).astype(jnp.bfloat16)

    def pack(m):
        mb = m.astype(jnp.bfloat16)
        a = jnp.dot(mb, pk_lo, preferred_element_type=jnp.float32)
        b = jnp.dot(mb, pk_hi, preferred_element_type=jnp.float32)
        return a.astype(jnp.int32) | (b.astype(jnp.int32) << 16)

    wp = pack(wf)
    sp = pack(sf)

    def shr1(a):
        return (a << 1) | jax.lax.shift_right_logical(_sr(a, 1), 31)

    def shl1(a):
        return jax.lax.shift_right_logical(a, 1) | (_sl(a, 1) << 31)

    se = shr1(sp)
    sw = shl1(sp)
    h3 = sp | se | sw
    dp = se | sw | _sd(h3, 1) | _su(h3, 1)

    wpt = wp.T
    dpt = dp.T
    for i in range(H):
        wtab_ref[i] = wpt[:, i:i + 1]
        dtab_ref[i] = dpt[:, i:i + 1]

    def row_body(i, p):
        w = wtab_ref[i]
        d = dtab_ref[i]
        wl0 = jnp.where(w == FULL, FULL, 0)
        wl1 = wl0 & _sd(wl0, 1)
        wl2 = wl1 & _sd(wl1, 2)
        wl3 = wl2 & _sd(wl2, 4)
        q = w
        for s in (1, 2, 4, 8, 16):
            q = q & ((q << s) | ((1 << s) - 1))
        c = (p | ((p << 1) | jax.lax.shift_right_logical(_sd(p, 1), 31))
               | (jax.lax.shift_right_logical(p, 1) | (_su(p, 1) << 31)))
        g0 = w & (c | d)
        f = _fill(w, g0)
        h = jax.lax.shift_right_logical(f, 31)
        h = h | (wl0 & _sd(h, 1))
        h = h | (wl1 & _sd(h, 2))
        h = h | (wl2 & _sd(h, 4))
        h = h | (wl3 & _sd(h, 8))
        cin = _sd(h, 1)
        p_new = f | ((0 - cin) & q)
        ptab_ref[i] = p_new
        return p_new

    jax.lax.fori_loop(0, H, row_body, jnp.zeros((NW, 1), jnp.int32))

    cols = [ptab_ref[i] for i in range(H)]
    ppt = jnp.concatenate(cols, axis=1)
    pp = ppt.T

    l2 = jax.lax.broadcasted_iota(jnp.int32, (NW, W), 0)
    j2 = jax.lax.broadcasted_iota(jnp.int32, (NW, W), 1)
    rept = jnp.where(l2 == (j2 // 32), 1, 0).astype(jnp.bfloat16)
    imgs = []
    for b in range(4):
        byte = jax.lax.shift_right_logical(pp, 8 * b) & 0xFF
        byteb = byte.astype(jnp.float32).astype(jnp.bfloat16)
        imgs.append(jnp.dot(byteb, rept, preferred_element_type=jnp.float32))
    bsel = (col % 32) // 8
    src = jnp.where(bsel == 0, imgs[0],
                    jnp.where(bsel == 1, imgs[1],
                              jnp.where(bsel == 2, imgs[2], imgs[3])))
    pbit = jax.lax.shift_right_logical(src.astype(jnp.int32), col % 8) & 1

    tx = jnp.where(weakb, WEAK, jnp.where(x >= hi, STRONG, 0.0))
    interior = jnp.logical_and(inrow, incol)
    o_ref[...] = jnp.where(
        jnp.logical_and(weakb, interior),
        jnp.where(pbit > 0, STRONG, 0.0),
        tx,
    )


@functools.partial(jax.jit)
def kernel(img):
    x = img.reshape(H, W)
    out = pl.pallas_call(
        _hyst_kernel,
        out_shape=jax.ShapeDtypeStruct((H, W), jnp.float32),
        scratch_shapes=[
            pltpu.VMEM((H, NW, 1), jnp.int32),
            pltpu.VMEM((H, NW, 1), jnp.int32),
            pltpu.VMEM((H, NW, 1), jnp.int32),
        ],
    )(x)
    return out[None, None, :, :]
